# baseline (device time: 48499 ns/iter reference)
import jax
import jax.numpy as jnp
from jax import lax
from jax.experimental import pallas as pl
from jax.experimental.pallas import tpu as pltpu

N_DEV = 4
B_LOC = 2
SQ = 256
SKV = 256
D_MODEL = 512
HQ_TOT = 16
HQ_LOC = 4
DH = 64
BLK = 64
SCALE = 0.125
NEG = -1e9

COMM_DTYPE = jnp.bfloat16

WQ_HALF = D_MODEL // 2
WO_HALF = (HQ_LOC * DH) // 2


def kernel(x, Wq, K_ext, V_ext, Wo):
    Wq_c = Wq.astype(COMM_DTYPE)
    Wo_c = Wo.astype(COMM_DTYPE)

    def body(x_ref, wq_ref, k_hbm, v_hbm, wo_ref, out_ref,
             khm, vhm, wq_comm, wo_comm, kv_sems, ssem, rsem):
        i = lax.axis_index("i")
        left = lax.rem(i + N_DEV - 1, N_DEV)
        right = lax.rem(i + 1, N_DEV)
        opp = lax.rem(i + 2, N_DEV)
        bg = i * B_LOC

        kv_copies = []
        for t, (src, dst) in enumerate(((k_hbm, khm), (v_hbm, vhm))):
            for h in range(HQ_TOT):
                c = pltpu.make_async_copy(
                    src.at[pl.ds(bg, B_LOC), :, h, :],
                    dst.at[pl.ds(h * B_LOC, B_LOC)],
                    kv_sems.at[t * HQ_TOT + h],
                )
                c.start()
                kv_copies.append(c)

        barrier = pltpu.get_barrier_semaphore()
        for nbr in (left, right):
            pl.semaphore_signal(barrier, inc=1, device_id=(nbr,),
                                device_id_type=pl.DeviceIdType.MESH)
        pl.semaphore_wait(barrier, 2)

        qb = lax.broadcasted_iota(jnp.int32, (SQ, SKV), 0) // BLK
        kb = lax.broadcasted_iota(jnp.int32, (SQ, SKV), 1) // BLK
        mask = kb <= qb

        def compute_group(g, wq_g, wo_g, init):
            wq_g = wq_g.astype(jnp.float32)
            wo_g = wo_g.astype(jnp.float32)
            x2 = x_ref[...].reshape(B_LOC * SQ, D_MODEL)
            q2 = jnp.dot(x2, wq_g, preferred_element_type=jnp.float32)
            ctx_rows = []
            for b in range(B_LOC):
                q = q2[b * SQ:(b + 1) * SQ]
                ctx_parts = []
                for hh in range(HQ_LOC):
                    qh = q[:, hh * DH:(hh + 1) * DH]
                    idx = (g * HQ_LOC + hh) * B_LOC + b
                    kh = khm[pl.ds(idx, 1)][0]
                    vh = vhm[pl.ds(idx, 1)][0]
                    s = lax.dot_general(
                        qh, kh, (((1,), (1,)), ((), ())),
                        preferred_element_type=jnp.float32) * SCALE
                    s = jnp.where(mask, s, NEG)
                    m = jnp.max(s, axis=1, keepdims=True)
                    e = jnp.exp(s - m)
                    w = e / jnp.sum(e, axis=1, keepdims=True)
                    ctx_parts.append(
                        jnp.dot(w, vh, preferred_element_type=jnp.float32))
                ctx_rows.append(jnp.concatenate(ctx_parts, axis=1))
            ctx2 = jnp.concatenate(ctx_rows, axis=0)
            part = jnp.dot(ctx2, wo_g, preferred_element_type=jnp.float32)
            part = part.reshape(B_LOC, SQ, D_MODEL)
            if init:
                out_ref[...] = part
            else:
                out_ref[...] = out_ref[...] + part

        def rc(src, dst, si, ri, dev):
            return pltpu.make_async_remote_copy(
                src_ref=src, dst_ref=dst,
                send_sem=ssem.at[si], recv_sem=rsem.at[ri],
                device_id=(dev,), device_id_type=pl.DeviceIdType.MESH)

        cR_wq = rc(wq_ref, wq_comm.at[0], 0, 0, right)
        cR_wo = rc(wo_ref, wo_comm.at[0], 1, 1, right)
        cL_wq = rc(wq_ref, wq_comm.at[1], 2, 2, left)
        cL_wo = rc(wo_ref, wo_comm.at[1], 3, 3, left)
        fR_wq = rc(wq_comm.at[0, pl.ds(0, WQ_HALF)],
                   wq_comm.at[2, pl.ds(0, WQ_HALF)], 4, 4, right)
        fR_wo = rc(wo_comm.at[0, pl.ds(0, WO_HALF)],
                   wo_comm.at[2, pl.ds(0, WO_HALF)], 5, 5, right)
        fL_wq = rc(wq_comm.at[1, pl.ds(WQ_HALF, WQ_HALF)],
                   wq_comm.at[2, pl.ds(WQ_HALF, WQ_HALF)], 6, 6, left)
        fL_wo = rc(wo_comm.at[1, pl.ds(WO_HALF, WO_HALF)],
                   wo_comm.at[2, pl.ds(WO_HALF, WO_HALF)], 7, 7, left)

        cR_wq.start()
        cR_wo.start()
        cL_wq.start()
        cL_wo.start()

        for c in kv_copies:
            c.wait()

        compute_group(i, wq_ref[...], wo_ref[...], init=True)

        cR_wq.wait_recv()
        cR_wo.wait_recv()
        fR_wq.start()
        fR_wo.start()
        cL_wq.wait_recv()
        cL_wo.wait_recv()
        fL_wq.start()
        fL_wo.start()

        compute_group(left, wq_comm[0], wo_comm[0], init=False)
        compute_group(right, wq_comm[1], wo_comm[1], init=False)

        fR_wq.wait_recv()
        fR_wo.wait_recv()
        fL_wq.wait_recv()
        fL_wo.wait_recv()
        compute_group(opp, wq_comm[2], wo_comm[2], init=False)

        for d in (cR_wq, cR_wo, cL_wq, cL_wo, fR_wq, fR_wo, fL_wq, fL_wo):
            d.wait_send()

    return pl.pallas_call(
        body,
        out_shape=jax.ShapeDtypeStruct((B_LOC, SQ, D_MODEL), jnp.float32),
        in_specs=[
            pl.BlockSpec(memory_space=pltpu.VMEM),
            pl.BlockSpec(memory_space=pltpu.VMEM),
            pl.BlockSpec(memory_space=pl.ANY),
            pl.BlockSpec(memory_space=pl.ANY),
            pl.BlockSpec(memory_space=pltpu.VMEM),
        ],
        out_specs=pl.BlockSpec(memory_space=pltpu.VMEM),
        scratch_shapes=[
            pltpu.VMEM((HQ_TOT * B_LOC, SKV, DH), jnp.float32),
            pltpu.VMEM((HQ_TOT * B_LOC, SKV, DH), jnp.float32),
            pltpu.VMEM((3, D_MODEL, HQ_LOC * DH), COMM_DTYPE),
            pltpu.VMEM((3, HQ_LOC * DH, D_MODEL), COMM_DTYPE),
            pltpu.SemaphoreType.DMA((2 * HQ_TOT,)),
            pltpu.SemaphoreType.DMA((8,)),
            pltpu.SemaphoreType.DMA((8,)),
        ],
        compiler_params=pltpu.CompilerParams(collective_id=0),
    )(x, Wq_c, K_ext, V_ext, Wo_c)


# device time: 24597 ns/iter; 1.9717x vs baseline; 1.9717x over previous
import jax
import jax.numpy as jnp
from jax import lax
from jax.experimental import pallas as pl
from jax.experimental.pallas import tpu as pltpu

N_DEV = 4
B_LOC = 2
SQ = 256
SKV = 256
D_MODEL = 512
HQ_TOT = 16
HQ_LOC = 4
DH = 64
BLK = 64
SCALE = 0.125
NEG = -1e9

COMM_DTYPE = jnp.bfloat16

WQ_HALF = D_MODEL // 2
WO_HALF = (HQ_LOC * DH) // 2


def kernel(x, Wq, K_ext, V_ext, Wo):
    my = lax.axis_index("i")
    K_loc = lax.dynamic_slice_in_dim(K_ext, my * B_LOC, B_LOC, axis=0)
    V_loc = lax.dynamic_slice_in_dim(V_ext, my * B_LOC, B_LOC, axis=0)
    K_t = jnp.transpose(K_loc, (2, 0, 1, 3)).reshape(HQ_TOT * B_LOC, SKV, DH)
    V_t = jnp.transpose(V_loc, (2, 0, 1, 3)).reshape(HQ_TOT * B_LOC, SKV, DH)
    Wq_c = Wq.astype(COMM_DTYPE)
    Wo_c = Wo.astype(COMM_DTYPE)

    def body(x_ref, wq_ref, khm, vhm, wo_ref, out_ref,
             wq_comm, wo_comm, ssem, rsem):
        i = lax.axis_index("i")
        left = lax.rem(i + N_DEV - 1, N_DEV)
        right = lax.rem(i + 1, N_DEV)
        opp = lax.rem(i + 2, N_DEV)

        barrier = pltpu.get_barrier_semaphore()
        for nbr in (left, right):
            pl.semaphore_signal(barrier, inc=1, device_id=(nbr,),
                                device_id_type=pl.DeviceIdType.MESH)
        pl.semaphore_wait(barrier, 2)

        qb = lax.broadcasted_iota(jnp.int32, (SQ, SKV), 0) // BLK
        kb = lax.broadcasted_iota(jnp.int32, (SQ, SKV), 1) // BLK
        mask = kb <= qb

        def compute_group(g, wq_g, wo_g, init):
            wq_g = wq_g.astype(jnp.float32)
            wo_g = wo_g.astype(jnp.float32)
            x2 = x_ref[...].reshape(B_LOC * SQ, D_MODEL)
            q2 = jnp.dot(x2, wq_g, preferred_element_type=jnp.float32)
            ctx_rows = []
            for b in range(B_LOC):
                q = q2[b * SQ:(b + 1) * SQ]
                ctx_parts = []
                for hh in range(HQ_LOC):
                    qh = q[:, hh * DH:(hh + 1) * DH]
                    idx = (g * HQ_LOC + hh) * B_LOC + b
                    kh = khm[pl.ds(idx, 1)][0]
                    vh = vhm[pl.ds(idx, 1)][0]
                    s = lax.dot_general(
                        qh, kh, (((1,), (1,)), ((), ())),
                        preferred_element_type=jnp.float32) * SCALE
                    s = jnp.where(mask, s, NEG)
                    m = jnp.max(s, axis=1, keepdims=True)
                    e = jnp.exp(s - m)
                    w = e / jnp.sum(e, axis=1, keepdims=True)
                    ctx_parts.append(
                        jnp.dot(w, vh, preferred_element_type=jnp.float32))
                ctx_rows.append(jnp.concatenate(ctx_parts, axis=1))
            ctx2 = jnp.concatenate(ctx_rows, axis=0)
            part = jnp.dot(ctx2, wo_g, preferred_element_type=jnp.float32)
            part = part.reshape(B_LOC, SQ, D_MODEL)
            if init:
                out_ref[...] = part
            else:
                out_ref[...] = out_ref[...] + part

        def rc(src, dst, si, ri, dev):
            return pltpu.make_async_remote_copy(
                src_ref=src, dst_ref=dst,
                send_sem=ssem.at[si], recv_sem=rsem.at[ri],
                device_id=(dev,), device_id_type=pl.DeviceIdType.MESH)

        cR_wq = rc(wq_ref, wq_comm.at[0], 0, 0, right)
        cR_wo = rc(wo_ref, wo_comm.at[0], 1, 1, right)
        cL_wq = rc(wq_ref, wq_comm.at[1], 2, 2, left)
        cL_wo = rc(wo_ref, wo_comm.at[1], 3, 3, left)
        fR_wq = rc(wq_comm.at[0, pl.ds(0, WQ_HALF)],
                   wq_comm.at[2, pl.ds(0, WQ_HALF)], 4, 4, right)
        fR_wo = rc(wo_comm.at[0, pl.ds(0, WO_HALF)],
                   wo_comm.at[2, pl.ds(0, WO_HALF)], 5, 5, right)
        fL_wq = rc(wq_comm.at[1, pl.ds(WQ_HALF, WQ_HALF)],
                   wq_comm.at[2, pl.ds(WQ_HALF, WQ_HALF)], 6, 6, left)
        fL_wo = rc(wo_comm.at[1, pl.ds(WO_HALF, WO_HALF)],
                   wo_comm.at[2, pl.ds(WO_HALF, WO_HALF)], 7, 7, left)

        cR_wq.start()
        cR_wo.start()
        cL_wq.start()
        cL_wo.start()

        compute_group(i, wq_ref[...], wo_ref[...], init=True)

        cR_wq.wait_recv()
        cR_wo.wait_recv()
        fR_wq.start()
        fR_wo.start()
        cL_wq.wait_recv()
        cL_wo.wait_recv()
        fL_wq.start()
        fL_wo.start()

        compute_group(left, wq_comm[0], wo_comm[0], init=False)
        compute_group(right, wq_comm[1], wo_comm[1], init=False)

        fR_wq.wait_recv()
        fR_wo.wait_recv()
        fL_wq.wait_recv()
        fL_wo.wait_recv()
        compute_group(opp, wq_comm[2], wo_comm[2], init=False)

        for d in (cR_wq, cR_wo, cL_wq, cL_wo, fR_wq, fR_wo, fL_wq, fL_wo):
            d.wait_send()

    return pl.pallas_call(
        body,
        out_shape=jax.ShapeDtypeStruct((B_LOC, SQ, D_MODEL), jnp.float32),
        in_specs=[pl.BlockSpec(memory_space=pltpu.VMEM)] * 5,
        out_specs=pl.BlockSpec(memory_space=pltpu.VMEM),
        scratch_shapes=[
            pltpu.VMEM((3, D_MODEL, HQ_LOC * DH), COMM_DTYPE),
            pltpu.VMEM((3, HQ_LOC * DH, D_MODEL), COMM_DTYPE),
            pltpu.SemaphoreType.DMA((8,)),
            pltpu.SemaphoreType.DMA((8,)),
        ],
        compiler_params=pltpu.CompilerParams(collective_id=0),
    )(x, Wq_c, K_t, V_t, Wo_c)


# device time: 21982 ns/iter; 2.2063x vs baseline; 1.1190x over previous
import jax
import jax.numpy as jnp
from jax import lax
from jax.experimental import pallas as pl
from jax.experimental.pallas import tpu as pltpu

N_DEV = 4
B_LOC = 2
SQ = 256
SKV = 256
D_MODEL = 512
HQ_TOT = 16
HQ_LOC = 4
DH = 64
BLK = 64
SCALE = 0.125
NEG = -1e9

COMM_DTYPE = jnp.bfloat16

WQ_HALF = D_MODEL // 2
WO_HALF = (HQ_LOC * DH) // 2


def kernel(x, Wq, K_ext, V_ext, Wo):
    my = lax.axis_index("i")
    K_loc = lax.dynamic_slice_in_dim(K_ext, my * B_LOC, B_LOC, axis=0)
    V_loc = lax.dynamic_slice_in_dim(V_ext, my * B_LOC, B_LOC, axis=0)
    K_t = jnp.transpose(K_loc, (2, 0, 1, 3)).reshape(HQ_TOT * B_LOC, SKV, DH)
    V_t = jnp.transpose(V_loc, (2, 0, 1, 3)).reshape(HQ_TOT * B_LOC, SKV, DH)
    Wq_c = Wq.astype(COMM_DTYPE)
    Wo_c = Wo.astype(COMM_DTYPE)

    def body(x_ref, wq_ref, khm, vhm, wo_ref, out_ref,
             wq_comm, wo_comm, ssem, rsem):
        i = lax.axis_index("i")
        left = lax.rem(i + N_DEV - 1, N_DEV)
        right = lax.rem(i + 1, N_DEV)
        opp = lax.rem(i + 2, N_DEV)

        barrier = pltpu.get_barrier_semaphore()
        for nbr in (left, right):
            pl.semaphore_signal(barrier, inc=1, device_id=(nbr,),
                                device_id_type=pl.DeviceIdType.MESH)
        pl.semaphore_wait(barrier, 2)

        qb = lax.broadcasted_iota(jnp.int32, (SQ, SKV), 0) // BLK
        kb = lax.broadcasted_iota(jnp.int32, (SQ, SKV), 1) // BLK
        mask = kb <= qb

        def attn_part(g, wq_g):
            wq_g = wq_g.astype(jnp.float32)
            x2 = x_ref[...].reshape(B_LOC * SQ, D_MODEL)
            q2 = jnp.dot(x2, wq_g, preferred_element_type=jnp.float32)
            ctx_rows = []
            for b in range(B_LOC):
                q = q2[b * SQ:(b + 1) * SQ]
                ctx_parts = []
                for hh in range(HQ_LOC):
                    qh = q[:, hh * DH:(hh + 1) * DH]
                    idx = (g * HQ_LOC + hh) * B_LOC + b
                    kh = khm[pl.ds(idx, 1)][0]
                    vh = vhm[pl.ds(idx, 1)][0]
                    s = lax.dot_general(
                        qh, kh, (((1,), (1,)), ((), ())),
                        preferred_element_type=jnp.float32) * SCALE
                    e = jnp.where(mask, jnp.exp(s), 0.0)
                    w = e / jnp.sum(e, axis=1, keepdims=True)
                    ctx_parts.append(
                        jnp.dot(w, vh, preferred_element_type=jnp.float32))
                ctx_rows.append(jnp.concatenate(ctx_parts, axis=1))
            return jnp.concatenate(ctx_rows, axis=0)

        def out_part(ctx2, wo_g, init):
            wo_g = wo_g.astype(jnp.float32)
            part = jnp.dot(ctx2, wo_g, preferred_element_type=jnp.float32)
            part = part.reshape(B_LOC, SQ, D_MODEL)
            if init:
                out_ref[...] = part
            else:
                out_ref[...] = out_ref[...] + part

        def rc(src, dst, si, ri, dev):
            return pltpu.make_async_remote_copy(
                src_ref=src, dst_ref=dst,
                send_sem=ssem.at[si], recv_sem=rsem.at[ri],
                device_id=(dev,), device_id_type=pl.DeviceIdType.MESH)

        cR_wq = rc(wq_ref, wq_comm.at[0], 0, 0, right)
        cR_wo = rc(wo_ref, wo_comm.at[0], 1, 1, right)
        cL_wq = rc(wq_ref, wq_comm.at[1], 2, 2, left)
        cL_wo = rc(wo_ref, wo_comm.at[1], 3, 3, left)
        fR_wq = rc(wq_comm.at[0, pl.ds(0, WQ_HALF)],
                   wq_comm.at[2, pl.ds(0, WQ_HALF)], 4, 4, right)
        fR_wo = rc(wo_comm.at[0, pl.ds(0, WO_HALF)],
                   wo_comm.at[2, pl.ds(0, WO_HALF)], 5, 5, right)
        fL_wq = rc(wq_comm.at[1, pl.ds(WQ_HALF, WQ_HALF)],
                   wq_comm.at[2, pl.ds(WQ_HALF, WQ_HALF)], 6, 6, left)
        fL_wo = rc(wo_comm.at[1, pl.ds(WO_HALF, WO_HALF)],
                   wo_comm.at[2, pl.ds(WO_HALF, WO_HALF)], 7, 7, left)

        cR_wq.start()
        cR_wo.start()
        cL_wq.start()
        cL_wo.start()

        ctx_own = attn_part(i, wq_ref[...])
        out_part(ctx_own, wo_ref[...], init=True)

        cR_wq.wait_recv()
        fR_wq.start()
        cL_wq.wait_recv()
        fL_wq.start()

        ctx_l = attn_part(left, wq_comm[0])
        cR_wo.wait_recv()
        fR_wo.start()
        out_part(ctx_l, wo_comm[0], init=False)

        ctx_r = attn_part(right, wq_comm[1])
        cL_wo.wait_recv()
        fL_wo.start()
        out_part(ctx_r, wo_comm[1], init=False)

        fR_wq.wait_recv()
        fL_wq.wait_recv()
        ctx_o = attn_part(opp, wq_comm[2])
        fR_wo.wait_recv()
        fL_wo.wait_recv()
        out_part(ctx_o, wo_comm[2], init=False)

        for d in (cR_wq, cR_wo, cL_wq, cL_wo, fR_wq, fR_wo, fL_wq, fL_wo):
            d.wait_send()

    return pl.pallas_call(
        body,
        out_shape=jax.ShapeDtypeStruct((B_LOC, SQ, D_MODEL), jnp.float32),
        in_specs=[pl.BlockSpec(memory_space=pltpu.VMEM)] * 5,
        out_specs=pl.BlockSpec(memory_space=pltpu.VMEM),
        scratch_shapes=[
            pltpu.VMEM((3, D_MODEL, HQ_LOC * DH), COMM_DTYPE),
            pltpu.VMEM((3, HQ_LOC * DH, D_MODEL), COMM_DTYPE),
            pltpu.SemaphoreType.DMA((8,)),
            pltpu.SemaphoreType.DMA((8,)),
        ],
        compiler_params=pltpu.CompilerParams(collective_id=0),
    )(x, Wq_c, K_t, V_t, Wo_c)
